# initial kernel scaffold (unmeasured)
import jax
import jax.numpy as jnp
from jax import lax
from jax.experimental import pallas as pl
from jax.experimental.pallas import tpu as pltpu

N_DEV = 8
SQ = 1024
SKV = 1024
HQ_LOCAL = 8
DH = 128
BLK = 64
SCALE = 0.08838834764831843


def kernel(x, Wq, K_ext, V_ext, Wo):
    def body(x_ref, wq_ref, k_ref, v_ref, wo_ref, out_ref,
             kstage, ksend, kv_comm, acc, rs_recv,
             copy_sem, scat_send_sem, scat_recv_sem,
             rs_send_sems, rs_recv_sems, ag_send_sems, ag_recv_sems):
        me = lax.axis_index("i")

        barrier = pltpu.get_barrier_semaphore()
        for d in range(N_DEV):
            @pl.when(me != d)
            def _():
                pl.semaphore_signal(
                    barrier, inc=1, device_id=(d,),
                    device_id_type=pl.DeviceIdType.MESH)
        pl.semaphore_wait(barrier, N_DEV - 1)

        @pl.when(me == 0)
        def _():
            for j in range(N_DEV):
                for t, src in ((0, k_ref), (1, v_ref)):
                    cp = pltpu.make_async_copy(
                        src.at[0, :, pl.ds(HQ_LOCAL * j, HQ_LOCAL), :],
                        kstage, copy_sem)
                    cp.start()
                    cp.wait()
                    ksend[t] = kstage[...].astype(jnp.bfloat16)
                if j == 0:
                    kv_comm[...] = ksend[...]
                else:
                    rdma = pltpu.make_async_remote_copy(
                        src_ref=ksend, dst_ref=kv_comm,
                        send_sem=scat_send_sem, recv_sem=scat_recv_sem,
                        device_id=(j,), device_id_type=pl.DeviceIdType.MESH)
                    rdma.start()
                    rdma.wait_send()

        @pl.when(me != 0)
        def _():
            recv = pltpu.make_async_remote_copy(
                src_ref=ksend, dst_ref=kv_comm,
                send_sem=scat_send_sem, recv_sem=scat_recv_sem,
                device_id=(0,), device_id_type=pl.DeviceIdType.MESH)
            recv.wait_recv()

        xb = x_ref[0].astype(jnp.bfloat16)
        wqb = wq_ref[...].astype(jnp.bfloat16)
        q = jnp.dot(xb, wqb, preferred_element_type=jnp.float32)
        qs = (q * SCALE).astype(jnp.bfloat16)

        rows = lax.broadcasted_iota(jnp.int32, (SQ, SKV), 0) // BLK
        cols = lax.broadcasted_iota(jnp.int32, (SQ, SKV), 1) // BLK
        neg = jnp.where(cols <= rows, 0.0, -1e9).astype(jnp.float32)

        ctx_heads = []
        for h in range(HQ_LOCAL):
            qh = qs[:, h * DH:(h + 1) * DH]
            kh = kv_comm[0, :, h, :]
            scores = lax.dot_general(
                qh, kh, (((1,), (1,)), ((), ())),
                preferred_element_type=jnp.float32) + neg
            mx = jnp.max(scores, axis=1, keepdims=True)
            w = jnp.exp(scores - mx)
            wn = (w / jnp.sum(w, axis=1, keepdims=True)).astype(jnp.bfloat16)
            vh = kv_comm[1, :, h, :]
            ctx_heads.append(
                jnp.dot(wn, vh, preferred_element_type=jnp.float32)
                .astype(jnp.bfloat16))
        ctx = jnp.concatenate(ctx_heads, axis=1)
        wob = wo_ref[...].astype(jnp.bfloat16)
        acc[...] = jnp.dot(ctx, wob, preferred_element_type=jnp.float32)

        z = me // 4
        r = me % 4
        y = r // 2
        xbit = (r % 2) ^ y
        px = me ^ 1
        py = me ^ 3
        pz = me ^ 4

        keep = 0
        for s, (p, size, bit) in enumerate(
                ((pz, 512, z), (py, 256, y), (px, 128, xbit))):
            send_off = keep + size * (1 - bit)
            keep = keep + size * bit
            rdma = pltpu.make_async_remote_copy(
                src_ref=acc.at[pl.ds(send_off, size)],
                dst_ref=rs_recv.at[pl.ds(0, size)],
                send_sem=rs_send_sems.at[s], recv_sem=rs_recv_sems.at[s],
                device_id=(p,), device_id_type=pl.DeviceIdType.MESH)
            rdma.start()
            rdma.wait()
            acc[pl.ds(keep, size)] = (
                acc[pl.ds(keep, size)] + rs_recv[pl.ds(0, size)])

        off = keep
        for s, (p, size, bit) in enumerate(
                ((px, 128, xbit), (py, 256, y), (pz, 512, z))):
            rdma = pltpu.make_async_remote_copy(
                src_ref=acc.at[pl.ds(off, size)],
                dst_ref=acc.at[pl.ds(off, size)],
                send_sem=ag_send_sems.at[s], recv_sem=ag_recv_sems.at[s],
                device_id=(p,), device_id_type=pl.DeviceIdType.MESH)
            rdma.start()
            rdma.wait()
            off = off - size * bit

        out_ref[0] = acc[...]

    return pl.pallas_call(
        body,
        out_shape=jax.ShapeDtypeStruct((1, SQ, SQ), jnp.float32),
        in_specs=[
            pl.BlockSpec(memory_space=pltpu.VMEM),
            pl.BlockSpec(memory_space=pltpu.VMEM),
            pl.BlockSpec(memory_space=pltpu.ANY),
            pl.BlockSpec(memory_space=pltpu.ANY),
            pl.BlockSpec(memory_space=pltpu.VMEM),
        ],
        out_specs=pl.BlockSpec(memory_space=pltpu.VMEM),
        scratch_shapes=[
            pltpu.VMEM((SKV, HQ_LOCAL, DH), jnp.float32),
            pltpu.VMEM((2, SKV, HQ_LOCAL, DH), jnp.bfloat16),
            pltpu.VMEM((2, SKV, HQ_LOCAL, DH), jnp.bfloat16),
            pltpu.VMEM((SQ, SQ), jnp.float32),
            pltpu.VMEM((512, SQ), jnp.float32),
            pltpu.SemaphoreType.DMA,
            pltpu.SemaphoreType.DMA,
            pltpu.SemaphoreType.DMA,
            pltpu.SemaphoreType.DMA((3,)),
            pltpu.SemaphoreType.DMA((3,)),
            pltpu.SemaphoreType.DMA((3,)),
            pltpu.SemaphoreType.DMA((3,)),
        ],
        compiler_params=pltpu.CompilerParams(
            collective_id=0,
            vmem_limit_bytes=100 * 1024 * 1024,
        ),
    )(x, Wq, K_ext, V_ext, Wo)


# baseline (device time: 487714 ns/iter reference)
import jax
import jax.numpy as jnp
from jax import lax
from jax.experimental import pallas as pl
from jax.experimental.pallas import tpu as pltpu

N_DEV = 8
SQ = 1024
SKV = 1024
HQ_LOCAL = 8
DH = 128
BLK = 64
SCALE = 0.08838834764831843


def kernel(x, Wq, K_ext, V_ext, Wo):
    def body(x_ref, wq_ref, k_ref, v_ref, wo_ref, out_ref,
             kstage, ksend, kv_comm, acc, rs0, rs1, rs2,
             copy_sem, scat_send_sem, scat_recv_sem,
             rs_send_sems, rs_recv_sems, ag_send_sems, ag_recv_sems):
        me = lax.axis_index("i")

        barrier = pltpu.get_barrier_semaphore()
        for d in range(N_DEV):
            @pl.when(me != d)
            def _():
                pl.semaphore_signal(
                    barrier, inc=1, device_id=(d,),
                    device_id_type=pl.DeviceIdType.MESH)
        pl.semaphore_wait(barrier, N_DEV - 1)

        @pl.when(me == 0)
        def _():
            for j in range(N_DEV):
                for t, src in ((0, k_ref), (1, v_ref)):
                    cp = pltpu.make_async_copy(
                        src.at[0, :, pl.ds(HQ_LOCAL * j, HQ_LOCAL), :],
                        kstage, copy_sem)
                    cp.start()
                    cp.wait()
                    ksend[t] = kstage[...].astype(jnp.bfloat16)
                if j == 0:
                    kv_comm[...] = ksend[...]
                else:
                    rdma = pltpu.make_async_remote_copy(
                        src_ref=ksend, dst_ref=kv_comm,
                        send_sem=scat_send_sem, recv_sem=scat_recv_sem,
                        device_id=(j,), device_id_type=pl.DeviceIdType.MESH)
                    rdma.start()
                    rdma.wait_send()

        @pl.when(me != 0)
        def _():
            recv = pltpu.make_async_remote_copy(
                src_ref=ksend, dst_ref=kv_comm,
                send_sem=scat_send_sem, recv_sem=scat_recv_sem,
                device_id=(0,), device_id_type=pl.DeviceIdType.MESH)
            recv.wait_recv()

        xb = x_ref[0].astype(jnp.bfloat16)
        wqb = wq_ref[...].astype(jnp.bfloat16)
        q = jnp.dot(xb, wqb, preferred_element_type=jnp.float32)
        qs = (q * SCALE).astype(jnp.bfloat16)

        rows = lax.broadcasted_iota(jnp.int32, (SQ, SKV), 0) // BLK
        cols = lax.broadcasted_iota(jnp.int32, (SQ, SKV), 1) // BLK
        neg = jnp.where(cols <= rows, 0.0, -1e9).astype(jnp.float32)

        ctx_heads = []
        for h in range(HQ_LOCAL):
            qh = qs[:, h * DH:(h + 1) * DH]
            kh = kv_comm[0, :, h, :]
            scores = lax.dot_general(
                qh, kh, (((1,), (1,)), ((), ())),
                preferred_element_type=jnp.float32) + neg
            mx = jnp.max(scores, axis=1, keepdims=True)
            w = jnp.exp(scores - mx)
            wn = (w / jnp.sum(w, axis=1, keepdims=True)).astype(jnp.bfloat16)
            vh = kv_comm[1, :, h, :]
            ctx_heads.append(
                jnp.dot(wn, vh, preferred_element_type=jnp.float32)
                .astype(jnp.bfloat16))
        ctx = jnp.concatenate(ctx_heads, axis=1)
        wob = wo_ref[...].astype(jnp.bfloat16)
        acc[...] = jnp.dot(ctx, wob, preferred_element_type=jnp.float32)

        z = me // 4
        r = me % 4
        y = r // 2
        xbit = (r % 2) ^ y
        px = me ^ 1
        py = me ^ 3
        pz = me ^ 4

        keep = 0
        for s, (p, size, bit, rbuf) in enumerate(
                ((pz, 512, z, rs0), (py, 256, y, rs1), (px, 128, xbit, rs2))):
            send_off = keep + size * (1 - bit)
            keep = keep + size * bit
            rdma = pltpu.make_async_remote_copy(
                src_ref=acc.at[pl.ds(send_off, size)],
                dst_ref=rbuf,
                send_sem=rs_send_sems.at[s], recv_sem=rs_recv_sems.at[s],
                device_id=(p,), device_id_type=pl.DeviceIdType.MESH)
            rdma.start()
            rdma.wait()
            acc[pl.ds(keep, size)] = acc[pl.ds(keep, size)] + rbuf[...]

        off = keep
        for s, (p, size, bit) in enumerate(
                ((px, 128, xbit), (py, 256, y), (pz, 512, z))):
            rdma = pltpu.make_async_remote_copy(
                src_ref=acc.at[pl.ds(off, size)],
                dst_ref=acc.at[pl.ds(off, size)],
                send_sem=ag_send_sems.at[s], recv_sem=ag_recv_sems.at[s],
                device_id=(p,), device_id_type=pl.DeviceIdType.MESH)
            rdma.start()
            rdma.wait()
            off = off - size * bit

        out_ref[0] = acc[...]

    return pl.pallas_call(
        body,
        out_shape=jax.ShapeDtypeStruct((1, SQ, SQ), jnp.float32),
        in_specs=[
            pl.BlockSpec(memory_space=pltpu.VMEM),
            pl.BlockSpec(memory_space=pltpu.VMEM),
            pl.BlockSpec(memory_space=pl.ANY),
            pl.BlockSpec(memory_space=pl.ANY),
            pl.BlockSpec(memory_space=pltpu.VMEM),
        ],
        out_specs=pl.BlockSpec(memory_space=pltpu.VMEM),
        scratch_shapes=[
            pltpu.VMEM((SKV, HQ_LOCAL, DH), jnp.float32),
            pltpu.VMEM((2, SKV, HQ_LOCAL, DH), jnp.bfloat16),
            pltpu.VMEM((2, SKV, HQ_LOCAL, DH), jnp.bfloat16),
            pltpu.VMEM((SQ, SQ), jnp.float32),
            pltpu.VMEM((512, SQ), jnp.float32),
            pltpu.VMEM((256, SQ), jnp.float32),
            pltpu.VMEM((128, SQ), jnp.float32),
            pltpu.SemaphoreType.DMA,
            pltpu.SemaphoreType.DMA,
            pltpu.SemaphoreType.DMA,
            pltpu.SemaphoreType.DMA((3,)),
            pltpu.SemaphoreType.DMA((3,)),
            pltpu.SemaphoreType.DMA((3,)),
            pltpu.SemaphoreType.DMA((3,)),
        ],
        compiler_params=pltpu.CompilerParams(
            collective_id=0,
            vmem_limit_bytes=100 * 1024 * 1024,
        ),
    )(x, Wq, K_ext, V_ext, Wo)


# device time: 285407 ns/iter; 1.7088x vs baseline; 1.7088x over previous
import jax
import jax.numpy as jnp
from jax import lax
from jax.experimental import pallas as pl
from jax.experimental.pallas import tpu as pltpu

N_DEV = 8
SQ = 1024
SKV = 1024
HQ_LOCAL = 8
DH = 128
BLK = 64
SCALE = 0.08838834764831843
N_SBUF = 3

SCHEDULE = (
    (2, 2, 1, 0),
    (7, 2, 3, 0),
    (5, 2, 4, 0),
    (1, 2, 1, -1),
    (6, 0, 3, 1),
    (6, 1, 4, 1),
    (3, 2, 3, -1),
    (4, 2, 4, -1),
)
RELAYS = {
    1: ((0, 2, 2),),
    3: ((0, 2, 7), (1, 0, 6)),
    4: ((0, 2, 5), (1, 1, 6)),
}


def kernel(x, Wq, K_ext, V_ext, Wo):
    def body(x_ref, wq_ref, k_ref, v_ref, wo_ref, out_ref,
             kstage, sbuf, relay, kv_comm, acc, rs0, rs1, rs2,
             copy_sem, sc_send_sems, kv_recv_sems,
             relay_recv_sems, relay_fwd_sems,
             rs_send_sems, rs_recv_sems, ag_send_sems, ag_recv_sems):
        me = lax.axis_index("i")

        def part_src(buf, parts):
            return buf if parts == 2 else buf.at[parts]

        barrier = pltpu.get_barrier_semaphore()
        for d in range(N_DEV):
            @pl.when(me != d)
            def _():
                pl.semaphore_signal(
                    barrier, inc=1, device_id=(d,),
                    device_id_type=pl.DeviceIdType.MESH)
        pl.semaphore_wait(barrier, N_DEV - 1)

        @pl.when(me == 0)
        def _():
            def stage(tgt, t, dst):
                cp = pltpu.make_async_copy(
                    (k_ref if t == 0 else v_ref)
                    .at[0, :, pl.ds(HQ_LOCAL * tgt, HQ_LOCAL), :],
                    kstage, copy_sem)
                cp.start()
                cp.wait()
                dst[...] = kstage[...].astype(jnp.bfloat16)

            pending = {}
            for idx, (tgt, parts, hop, rslot) in enumerate(SCHEDULE):
                b = idx % N_SBUF
                if b in pending:
                    pending[b].wait_send()
                for t in ((0, 1) if parts == 2 else (parts,)):
                    stage(tgt, t, sbuf.at[b, t])
                if hop == tgt:
                    dst, rsem = kv_comm, kv_recv_sems.at[0]
                else:
                    dst = relay.at[rslot]
                    rsem = relay_recv_sems.at[rslot]
                rdma = pltpu.make_async_remote_copy(
                    src_ref=part_src(sbuf.at[b], parts),
                    dst_ref=part_src(dst, parts),
                    send_sem=sc_send_sems.at[b], recv_sem=rsem,
                    device_id=(hop,), device_id_type=pl.DeviceIdType.MESH)
                rdma.start()
                pending[b] = rdma
            for t in (0, 1):
                stage(0, t, kv_comm.at[t])
            for b in pending:
                pending[b].wait_send()

        for dev, duties in RELAYS.items():
            @pl.when(me == dev)
            def _(duties=duties):
                for slot, parts, ftgt in duties:
                    rx = pltpu.make_async_remote_copy(
                        src_ref=part_src(sbuf.at[0], parts),
                        dst_ref=part_src(relay.at[slot], parts),
                        send_sem=sc_send_sems.at[0],
                        recv_sem=relay_recv_sems.at[slot],
                        device_id=(0,), device_id_type=pl.DeviceIdType.MESH)
                    rx.wait_recv()
                    sem = kv_recv_sems.at[parts if ftgt == 6 else 0]
                    fwd = pltpu.make_async_remote_copy(
                        src_ref=part_src(relay.at[slot], parts),
                        dst_ref=part_src(kv_comm, parts),
                        send_sem=relay_fwd_sems.at[slot], recv_sem=sem,
                        device_id=(ftgt,), device_id_type=pl.DeviceIdType.MESH)
                    fwd.start()
                for slot, parts, _ in duties:
                    pltpu.make_async_remote_copy(
                        src_ref=part_src(relay.at[slot], parts),
                        dst_ref=part_src(kv_comm, parts),
                        send_sem=relay_fwd_sems.at[slot],
                        recv_sem=kv_recv_sems.at[0],
                        device_id=(0,),
                        device_id_type=pl.DeviceIdType.MESH).wait_send()

        @pl.when((me != 0) & (me != 6))
        def _():
            pltpu.make_async_remote_copy(
                src_ref=sbuf.at[0], dst_ref=kv_comm,
                send_sem=sc_send_sems.at[0], recv_sem=kv_recv_sems.at[0],
                device_id=(0,),
                device_id_type=pl.DeviceIdType.MESH).wait_recv()

        @pl.when(me == 6)
        def _():
            for t in (0, 1):
                pltpu.make_async_remote_copy(
                    src_ref=sbuf.at[0, t], dst_ref=kv_comm.at[t],
                    send_sem=sc_send_sems.at[0], recv_sem=kv_recv_sems.at[t],
                    device_id=(0,),
                    device_id_type=pl.DeviceIdType.MESH).wait_recv()

        xb = x_ref[0].astype(jnp.bfloat16)
        wqb = wq_ref[...].astype(jnp.bfloat16)
        q = jnp.dot(xb, wqb, preferred_element_type=jnp.float32)
        qs = (q * SCALE).astype(jnp.bfloat16)

        rows = lax.broadcasted_iota(jnp.int32, (SQ, SKV), 0) // BLK
        cols = lax.broadcasted_iota(jnp.int32, (SQ, SKV), 1) // BLK
        neg = jnp.where(cols <= rows, 0.0, -1e9).astype(jnp.float32)

        ctx_heads = []
        for h in range(HQ_LOCAL):
            qh = qs[:, h * DH:(h + 1) * DH]
            kh = kv_comm[0, :, h, :]
            scores = lax.dot_general(
                qh, kh, (((1,), (1,)), ((), ())),
                preferred_element_type=jnp.float32) + neg
            mx = jnp.max(scores, axis=1, keepdims=True)
            w = jnp.exp(scores - mx)
            wn = (w / jnp.sum(w, axis=1, keepdims=True)).astype(jnp.bfloat16)
            vh = kv_comm[1, :, h, :]
            ctx_heads.append(
                jnp.dot(wn, vh, preferred_element_type=jnp.float32)
                .astype(jnp.bfloat16))
        ctx = jnp.concatenate(ctx_heads, axis=1)
        wob = wo_ref[...].astype(jnp.bfloat16)
        acc[...] = jnp.dot(ctx, wob, preferred_element_type=jnp.float32)

        z = me // 4
        r = me % 4
        y = r // 2
        xbit = (r % 2) ^ y
        px = me ^ 1
        py = me ^ 3
        pz = me ^ 4

        keep = 0
        for s, (p, size, bit, rbuf) in enumerate(
                ((pz, 512, z, rs0), (py, 256, y, rs1), (px, 128, xbit, rs2))):
            send_off = keep + size * (1 - bit)
            keep = keep + size * bit
            rdma = pltpu.make_async_remote_copy(
                src_ref=acc.at[pl.ds(send_off, size)],
                dst_ref=rbuf,
                send_sem=rs_send_sems.at[s], recv_sem=rs_recv_sems.at[s],
                device_id=(p,), device_id_type=pl.DeviceIdType.MESH)
            rdma.start()
            rdma.wait()
            acc[pl.ds(keep, size)] = acc[pl.ds(keep, size)] + rbuf[...]

        off = keep
        for s, (p, size, bit) in enumerate(
                ((px, 128, xbit), (py, 256, y), (pz, 512, z))):
            rdma = pltpu.make_async_remote_copy(
                src_ref=acc.at[pl.ds(off, size)],
                dst_ref=acc.at[pl.ds(off, size)],
                send_sem=ag_send_sems.at[s], recv_sem=ag_recv_sems.at[s],
                device_id=(p,), device_id_type=pl.DeviceIdType.MESH)
            rdma.start()
            rdma.wait()
            off = off - size * bit

        cp = pltpu.make_async_copy(acc, out_ref.at[0], copy_sem)
        cp.start()
        cp.wait()

    return pl.pallas_call(
        body,
        out_shape=jax.ShapeDtypeStruct((1, SQ, SQ), jnp.float32),
        in_specs=[
            pl.BlockSpec(memory_space=pltpu.VMEM),
            pl.BlockSpec(memory_space=pltpu.VMEM),
            pl.BlockSpec(memory_space=pl.ANY),
            pl.BlockSpec(memory_space=pl.ANY),
            pl.BlockSpec(memory_space=pltpu.VMEM),
        ],
        out_specs=pl.BlockSpec(memory_space=pl.ANY),
        scratch_shapes=[
            pltpu.VMEM((SKV, HQ_LOCAL, DH), jnp.float32),
            pltpu.VMEM((N_SBUF, 2, SKV, HQ_LOCAL, DH), jnp.bfloat16),
            pltpu.VMEM((2, 2, SKV, HQ_LOCAL, DH), jnp.bfloat16),
            pltpu.VMEM((2, SKV, HQ_LOCAL, DH), jnp.bfloat16),
            pltpu.VMEM((SQ, SQ), jnp.float32),
            pltpu.VMEM((512, SQ), jnp.float32),
            pltpu.VMEM((256, SQ), jnp.float32),
            pltpu.VMEM((128, SQ), jnp.float32),
            pltpu.SemaphoreType.DMA,
            pltpu.SemaphoreType.DMA((N_SBUF,)),
            pltpu.SemaphoreType.DMA((2,)),
            pltpu.SemaphoreType.DMA((2,)),
            pltpu.SemaphoreType.DMA((2,)),
            pltpu.SemaphoreType.DMA((3,)),
            pltpu.SemaphoreType.DMA((3,)),
            pltpu.SemaphoreType.DMA((3,)),
            pltpu.SemaphoreType.DMA((3,)),
        ],
        compiler_params=pltpu.CompilerParams(
            collective_id=0,
            vmem_limit_bytes=100 * 1024 * 1024,
        ),
    )(x, Wq, K_ext, V_ext, Wo)


# device time: 243073 ns/iter; 2.0065x vs baseline; 1.1742x over previous
import jax
import jax.numpy as jnp
from jax import lax
from jax.experimental import pallas as pl
from jax.experimental.pallas import tpu as pltpu

N_DEV = 8
SQ = 1024
SKV = 1024
HQ_LOCAL = 8
DH = 128
BLK = 64
SCALE = 0.08838834764831843
N_SBUF = 3

SCHEDULE = (
    (2, 2, 1, 0),
    (7, 2, 3, 0),
    (5, 2, 4, 0),
    (1, 2, 1, -1),
    (6, 0, 3, 1),
    (6, 1, 4, 1),
    (3, 2, 3, -1),
    (4, 2, 4, -1),
)
FINAL_PENDING = {0: 2, 1: 2, 2: 1}
RELAYS = {
    1: ((0, 2, 2),),
    3: ((0, 2, 7), (1, 0, 6)),
    4: ((0, 2, 5), (1, 1, 6)),
}


def kernel(x, Wq, K_ext, V_ext, Wo):
    def body(x_ref, wq_ref, k_ref, v_ref, wo_ref, out_ref,
             kstage, sbuf, relay, kv_comm, acc, accb, ctxr,
             ss0, ss1, ss2, rs0, rs1, rs2,
             copy_sem, sc_send_sems, kv_recv_sems,
             relay_recv_sems, relay_fwd_sems,
             rs_send_sems, rs_recv_sems, ag_send_sems, ag_recv_sems):
        me = lax.axis_index("i")

        def part_src(buf, parts):
            return buf if parts == 2 else buf.at[parts]

        barrier = pltpu.get_barrier_semaphore()
        for d in range(N_DEV):
            @pl.when(me != d)
            def _():
                pl.semaphore_signal(
                    barrier, inc=1, device_id=(d,),
                    device_id_type=pl.DeviceIdType.MESH)
        pl.semaphore_wait(barrier, N_DEV - 1)

        @pl.when(me == 0)
        def _():
            def stage(tgt, t, dst):
                for half in (0, 1):
                    cp = pltpu.make_async_copy(
                        (k_ref if t == 0 else v_ref)
                        .at[0, pl.ds(512 * half, 512),
                            pl.ds(HQ_LOCAL * tgt, HQ_LOCAL), :],
                        kstage, copy_sem)
                    cp.start()
                    cp.wait()
                    dst[pl.ds(512 * half, 512)] = (
                        kstage[...].astype(jnp.bfloat16))

            used = {}
            for idx, (tgt, parts, hop, rslot) in enumerate(SCHEDULE):
                b = idx % N_SBUF
                if b in used:
                    pltpu.make_async_remote_copy(
                        src_ref=part_src(sbuf.at[b], used[b]),
                        dst_ref=part_src(kv_comm, used[b]),
                        send_sem=sc_send_sems.at[b],
                        recv_sem=kv_recv_sems.at[0], device_id=(0,),
                        device_id_type=pl.DeviceIdType.MESH).wait_send()
                for t in ((0, 1) if parts == 2 else (parts,)):
                    stage(tgt, t, sbuf.at[b, t])
                if hop == tgt:
                    dstr, rsem = part_src(kv_comm, parts), kv_recv_sems.at[0]
                elif parts == 2:
                    dstr = relay.at[pl.ds(0, 2)]
                    rsem = relay_recv_sems.at[0]
                else:
                    dstr = relay.at[2]
                    rsem = relay_recv_sems.at[1]
                rdma = pltpu.make_async_remote_copy(
                    src_ref=part_src(sbuf.at[b], parts),
                    dst_ref=dstr,
                    send_sem=sc_send_sems.at[b], recv_sem=rsem,
                    device_id=(hop,), device_id_type=pl.DeviceIdType.MESH)
                rdma.start()
                used[b] = parts
            for t in (0, 1):
                stage(0, t, kv_comm.at[t])

        xb = x_ref[0].astype(jnp.bfloat16)
        wqb = wq_ref[...].astype(jnp.bfloat16)
        q = jnp.dot(xb, wqb, preferred_element_type=jnp.float32)
        qs = (q * SCALE).astype(jnp.bfloat16)

        rows = lax.broadcasted_iota(jnp.int32, (SQ, SKV), 0) // BLK
        cols = lax.broadcasted_iota(jnp.int32, (SQ, SKV), 1) // BLK
        neg = jnp.where(cols <= rows, 0.0, -1e9).astype(jnp.float32)
        wob = wo_ref[...].astype(jnp.bfloat16)

        for dev, duties in RELAYS.items():
            @pl.when(me == dev)
            def _(duties=duties):
                def rsrc(parts):
                    return (relay.at[pl.ds(0, 2)] if parts == 2
                            else relay.at[2])

                for slot, parts, ftgt in duties:
                    rx = pltpu.make_async_remote_copy(
                        src_ref=part_src(sbuf.at[0], parts),
                        dst_ref=rsrc(parts),
                        send_sem=sc_send_sems.at[0],
                        recv_sem=relay_recv_sems.at[0 if parts == 2 else 1],
                        device_id=(0,), device_id_type=pl.DeviceIdType.MESH)
                    rx.wait_recv()
                    sem = kv_recv_sems.at[parts if ftgt == 6 else 0]
                    fwd = pltpu.make_async_remote_copy(
                        src_ref=rsrc(parts),
                        dst_ref=part_src(kv_comm, parts),
                        send_sem=relay_fwd_sems.at[slot], recv_sem=sem,
                        device_id=(ftgt,), device_id_type=pl.DeviceIdType.MESH)
                    fwd.start()
                for slot, parts, _ in duties:
                    pltpu.make_async_remote_copy(
                        src_ref=rsrc(parts),
                        dst_ref=part_src(kv_comm, parts),
                        send_sem=relay_fwd_sems.at[slot],
                        recv_sem=kv_recv_sems.at[0],
                        device_id=(0,),
                        device_id_type=pl.DeviceIdType.MESH).wait_send()

        @pl.when((me != 0) & (me != 6))
        def _():
            pltpu.make_async_remote_copy(
                src_ref=sbuf.at[0], dst_ref=kv_comm,
                send_sem=sc_send_sems.at[0], recv_sem=kv_recv_sems.at[0],
                device_id=(0,),
                device_id_type=pl.DeviceIdType.MESH).wait_recv()

        @pl.when(me == 6)
        def _():
            for t in (0, 1):
                pltpu.make_async_remote_copy(
                    src_ref=sbuf.at[0, t], dst_ref=kv_comm.at[t],
                    send_sem=sc_send_sems.at[0], recv_sem=kv_recv_sems.at[t],
                    device_id=(0,),
                    device_id_type=pl.DeviceIdType.MESH).wait_recv()

        ctx_heads = []
        for h in range(HQ_LOCAL):
            qh = qs[:, h * DH:(h + 1) * DH]
            kh = kv_comm[0, :, h, :]
            scores = lax.dot_general(
                qh, kh, (((1,), (1,)), ((), ())),
                preferred_element_type=jnp.float32) + neg
            e = jnp.exp(scores)
            w = e.astype(jnp.bfloat16)
            s = jnp.sum(e, axis=1, keepdims=True)
            vh = kv_comm[1, :, h, :]
            ctx_heads.append(
                (jnp.dot(w, vh, preferred_element_type=jnp.float32) / s)
                .astype(jnp.bfloat16))
        ctxr[...] = jnp.concatenate(ctx_heads, axis=1)

        z = me // 4
        r = me % 4
        y = r // 2
        xbit = (r % 2) ^ y
        px = me ^ 1
        py = me ^ 3
        pz = me ^ 4

        send_off = 512 * (1 - z)
        keep = 512 * z
        ps = jnp.dot(ctxr[pl.ds(send_off, 512)], wob,
                     preferred_element_type=jnp.float32)
        ss0[...] = ps.astype(jnp.bfloat16)
        rdma_z = pltpu.make_async_remote_copy(
            src_ref=ss0, dst_ref=rs0,
            send_sem=rs_send_sems.at[0], recv_sem=rs_recv_sems.at[0],
            device_id=(pz,), device_id_type=pl.DeviceIdType.MESH)
        rdma_z.start()
        acc[pl.ds(keep, 512)] = jnp.dot(
            ctxr[pl.ds(keep, 512)], wob,
            preferred_element_type=jnp.float32)
        rdma_z.wait()
        acc[pl.ds(keep, 512)] = (
            acc[pl.ds(keep, 512)] + rs0[...].astype(jnp.float32))

        for s, (p, size, bit, sbuf_rs, rbuf) in enumerate(
                ((py, 256, y, ss1, rs1), (px, 128, xbit, ss2, rs2)), start=1):
            send_off = keep + size * (1 - bit)
            keep = keep + size * bit
            sbuf_rs[...] = acc[pl.ds(send_off, size)].astype(jnp.bfloat16)
            rdma = pltpu.make_async_remote_copy(
                src_ref=sbuf_rs, dst_ref=rbuf,
                send_sem=rs_send_sems.at[s], recv_sem=rs_recv_sems.at[s],
                device_id=(p,), device_id_type=pl.DeviceIdType.MESH)
            rdma.start()
            rdma.wait()
            acc[pl.ds(keep, size)] = (
                acc[pl.ds(keep, size)] + rbuf[...].astype(jnp.float32))

        accb[pl.ds(keep, 128)] = acc[pl.ds(keep, 128)].astype(jnp.bfloat16)
        off = keep
        for s, (p, size, bit) in enumerate(
                ((px, 128, xbit), (py, 256, y), (pz, 512, z))):
            rdma = pltpu.make_async_remote_copy(
                src_ref=accb.at[pl.ds(off, size)],
                dst_ref=accb.at[pl.ds(off, size)],
                send_sem=ag_send_sems.at[s], recv_sem=ag_recv_sems.at[s],
                device_id=(p,), device_id_type=pl.DeviceIdType.MESH)
            rdma.start()
            rdma.wait()
            off = off - size * bit

        @pl.when(me == 0)
        def _():
            for b, parts in FINAL_PENDING.items():
                pltpu.make_async_remote_copy(
                    src_ref=part_src(sbuf.at[b], parts),
                    dst_ref=part_src(kv_comm, parts),
                    send_sem=sc_send_sems.at[b], recv_sem=kv_recv_sems.at[0],
                    device_id=(0,),
                    device_id_type=pl.DeviceIdType.MESH).wait_send()

        acc[...] = accb[...].astype(jnp.float32)
        cp = pltpu.make_async_copy(acc, out_ref.at[0], copy_sem)
        cp.start()
        cp.wait()

    return pl.pallas_call(
        body,
        out_shape=jax.ShapeDtypeStruct((1, SQ, SQ), jnp.float32),
        in_specs=[
            pl.BlockSpec(memory_space=pltpu.VMEM),
            pl.BlockSpec(memory_space=pltpu.VMEM),
            pl.BlockSpec(memory_space=pl.ANY),
            pl.BlockSpec(memory_space=pl.ANY),
            pl.BlockSpec(memory_space=pltpu.VMEM),
        ],
        out_specs=pl.BlockSpec(memory_space=pl.ANY),
        scratch_shapes=[
            pltpu.VMEM((512, HQ_LOCAL, DH), jnp.float32),
            pltpu.VMEM((N_SBUF, 2, SKV, HQ_LOCAL, DH), jnp.bfloat16),
            pltpu.VMEM((3, SKV, HQ_LOCAL, DH), jnp.bfloat16),
            pltpu.VMEM((2, SKV, HQ_LOCAL, DH), jnp.bfloat16),
            pltpu.VMEM((SQ, SQ), jnp.float32),
            pltpu.VMEM((SQ, SQ), jnp.bfloat16),
            pltpu.VMEM((SQ, SQ), jnp.bfloat16),
            pltpu.VMEM((512, SQ), jnp.bfloat16),
            pltpu.VMEM((256, SQ), jnp.bfloat16),
            pltpu.VMEM((128, SQ), jnp.bfloat16),
            pltpu.VMEM((512, SQ), jnp.bfloat16),
            pltpu.VMEM((256, SQ), jnp.bfloat16),
            pltpu.VMEM((128, SQ), jnp.bfloat16),
            pltpu.SemaphoreType.DMA,
            pltpu.SemaphoreType.DMA((N_SBUF,)),
            pltpu.SemaphoreType.DMA((2,)),
            pltpu.SemaphoreType.DMA((2,)),
            pltpu.SemaphoreType.DMA((2,)),
            pltpu.SemaphoreType.DMA((3,)),
            pltpu.SemaphoreType.DMA((3,)),
            pltpu.SemaphoreType.DMA((3,)),
            pltpu.SemaphoreType.DMA((3,)),
        ],
        compiler_params=pltpu.CompilerParams(
            collective_id=0,
            vmem_limit_bytes=100 * 1024 * 1024,
        ),
    )(x, Wq, K_ext, V_ext, Wo)


# device time: 204239 ns/iter; 2.3880x vs baseline; 1.1901x over previous
import jax
import jax.numpy as jnp
from jax import lax
from jax.experimental import pallas as pl
from jax.experimental.pallas import tpu as pltpu

N_DEV = 8
SQ = 1024
SKV = 1024
HQ_LOCAL = 8
DH = 128
BLK = 64
SCALE = 0.08838834764831843
N_SBUF = 4

SCHEDULE = (
    (0, 2, 1, 0),
    (0, 7, 3, 0),
    (1, 5, 4, 0),
    (1, 2, 1, 1),
    (1, 7, 3, 1),
    (1, 6, 4, 1),
    (0, 5, 1, 2),
    (0, 6, 3, 2),
    (0, 4, 4, -1),
    (0, 1, 1, -1),
    (0, 3, 3, -1),
    (1, 4, 4, -1),
    (1, 1, 1, -1),
    (1, 3, 3, -1),
)
RELAYS = {
    1: ((0, 0, 2), (1, 1, 2), (2, 0, 5)),
    3: ((0, 0, 7), (1, 1, 7), (2, 0, 6)),
    4: ((0, 1, 5), (1, 1, 6)),
}


def kernel(x, Wq, K_ext, V_ext, Wo):
    def body(x_ref, wq_ref, k_ref, v_ref, wo_ref, out_ref,
             kstage, sbuf, relay, kv_comm, acc, accb, ctxr,
             ss0, ss1, ss2, rs0, rs1, rs2,
             copy_sem, sc_send_sems, kv_recv_sems,
             relay_recv_sems, relay_fwd_sems,
             rs_send_sems, rs_recv_sems, ag_send_sems, ag_recv_sems):
        me = lax.axis_index("i")

        barrier = pltpu.get_barrier_semaphore()
        for d in range(N_DEV):
            @pl.when(me != d)
            def _():
                pl.semaphore_signal(
                    barrier, inc=1, device_id=(d,),
                    device_id_type=pl.DeviceIdType.MESH)
        pl.semaphore_wait(barrier, N_DEV - 1)

        @pl.when(me == 0)
        def _():
            def stage(tgt, t, dst):
                for half in (0, 1):
                    cp = pltpu.make_async_copy(
                        (k_ref if t == 0 else v_ref)
                        .at[0, pl.ds(512 * half, 512),
                            pl.ds(HQ_LOCAL * tgt, HQ_LOCAL), :],
                        kstage, copy_sem)
                    cp.start()
                    cp.wait()
                    dst[pl.ds(512 * half, 512)] = (
                        kstage[...].astype(jnp.bfloat16))

            used = set()
            for idx, (t, tgt, hop, rslot) in enumerate(SCHEDULE):
                b = idx % N_SBUF
                if b in used:
                    pltpu.make_async_remote_copy(
                        src_ref=sbuf.at[b], dst_ref=kv_comm.at[0],
                        send_sem=sc_send_sems.at[b],
                        recv_sem=kv_recv_sems.at[0], device_id=(0,),
                        device_id_type=pl.DeviceIdType.MESH).wait_send()
                stage(tgt, t, sbuf.at[b])
                if hop == tgt:
                    dstr, rsem = kv_comm.at[t], kv_recv_sems.at[t]
                else:
                    dstr, rsem = relay.at[rslot], relay_recv_sems.at[rslot]
                rdma = pltpu.make_async_remote_copy(
                    src_ref=sbuf.at[b], dst_ref=dstr,
                    send_sem=sc_send_sems.at[b], recv_sem=rsem,
                    device_id=(hop,), device_id_type=pl.DeviceIdType.MESH)
                rdma.start()
                used.add(b)
            for t in (0, 1):
                stage(0, t, kv_comm.at[t])

        xb = x_ref[0].astype(jnp.bfloat16)
        wqb = wq_ref[...].astype(jnp.bfloat16)
        q = jnp.dot(xb, wqb, preferred_element_type=jnp.float32)
        qs = (q * SCALE).astype(jnp.bfloat16)

        rows = lax.broadcasted_iota(jnp.int32, (SQ, SKV), 0) // BLK
        cols = lax.broadcasted_iota(jnp.int32, (SQ, SKV), 1) // BLK
        neg = jnp.where(cols <= rows, 0.0, -1e9).astype(jnp.float32)
        wob = wo_ref[...].astype(jnp.bfloat16)

        for dev, duties in RELAYS.items():
            @pl.when(me == dev)
            def _(duties=duties):
                for slot, t, ftgt in duties:
                    rx = pltpu.make_async_remote_copy(
                        src_ref=sbuf.at[0], dst_ref=relay.at[slot],
                        send_sem=sc_send_sems.at[0],
                        recv_sem=relay_recv_sems.at[slot],
                        device_id=(0,), device_id_type=pl.DeviceIdType.MESH)
                    rx.wait_recv()
                    fwd = pltpu.make_async_remote_copy(
                        src_ref=relay.at[slot], dst_ref=kv_comm.at[t],
                        send_sem=relay_fwd_sems.at[slot],
                        recv_sem=kv_recv_sems.at[t],
                        device_id=(ftgt,), device_id_type=pl.DeviceIdType.MESH)
                    fwd.start()
                for slot, t, _ in duties:
                    pltpu.make_async_remote_copy(
                        src_ref=relay.at[slot], dst_ref=kv_comm.at[t],
                        send_sem=relay_fwd_sems.at[slot],
                        recv_sem=kv_recv_sems.at[0],
                        device_id=(0,),
                        device_id_type=pl.DeviceIdType.MESH).wait_send()

        @pl.when(me != 0)
        def _():
            for t in (0, 1):
                pltpu.make_async_remote_copy(
                    src_ref=sbuf.at[0], dst_ref=kv_comm.at[t],
                    send_sem=sc_send_sems.at[0], recv_sem=kv_recv_sems.at[t],
                    device_id=(0,),
                    device_id_type=pl.DeviceIdType.MESH).wait_recv()

        ctx_heads = []
        for h in range(HQ_LOCAL):
            qh = qs[:, h * DH:(h + 1) * DH]
            kh = kv_comm[0, :, h, :]
            scores = lax.dot_general(
                qh, kh, (((1,), (1,)), ((), ())),
                preferred_element_type=jnp.float32) + neg
            e = jnp.exp(scores)
            w = e.astype(jnp.bfloat16)
            s = jnp.sum(e, axis=1, keepdims=True)
            vh = kv_comm[1, :, h, :]
            ctx_heads.append(
                (jnp.dot(w, vh, preferred_element_type=jnp.float32) / s)
                .astype(jnp.bfloat16))
        ctxr[...] = jnp.concatenate(ctx_heads, axis=1)

        z = me // 4
        r = me % 4
        y = r // 2
        xbit = (r % 2) ^ y
        px = me ^ 1
        py = me ^ 3
        pz = me ^ 4

        send_off = 512 * (1 - z)
        keep = 512 * z
        ps = jnp.dot(ctxr[pl.ds(send_off, 512)], wob,
                     preferred_element_type=jnp.float32)
        ss0[...] = ps.astype(jnp.bfloat16)
        rdma_z = pltpu.make_async_remote_copy(
            src_ref=ss0, dst_ref=rs0,
            send_sem=rs_send_sems.at[0], recv_sem=rs_recv_sems.at[0],
            device_id=(pz,), device_id_type=pl.DeviceIdType.MESH)
        rdma_z.start()
        acc[pl.ds(keep, 512)] = jnp.dot(
            ctxr[pl.ds(keep, 512)], wob,
            preferred_element_type=jnp.float32)
        rdma_z.wait()
        acc[pl.ds(keep, 512)] = (
            acc[pl.ds(keep, 512)] + rs0[...].astype(jnp.float32))

        for s, (p, size, bit, sbuf_rs, rbuf) in enumerate(
                ((py, 256, y, ss1, rs1), (px, 128, xbit, ss2, rs2)), start=1):
            send_off = keep + size * (1 - bit)
            keep = keep + size * bit
            sbuf_rs[...] = acc[pl.ds(send_off, size)].astype(jnp.bfloat16)
            rdma = pltpu.make_async_remote_copy(
                src_ref=sbuf_rs, dst_ref=rbuf,
                send_sem=rs_send_sems.at[s], recv_sem=rs_recv_sems.at[s],
                device_id=(p,), device_id_type=pl.DeviceIdType.MESH)
            rdma.start()
            rdma.wait()
            acc[pl.ds(keep, size)] = (
                acc[pl.ds(keep, size)] + rbuf[...].astype(jnp.float32))

        accb[pl.ds(keep, 128)] = acc[pl.ds(keep, 128)].astype(jnp.bfloat16)
        off = keep
        for s, (p, size, bit) in enumerate(
                ((px, 128, xbit), (py, 256, y), (pz, 512, z))):
            rdma = pltpu.make_async_remote_copy(
                src_ref=accb.at[pl.ds(off, size)],
                dst_ref=accb.at[pl.ds(off, size)],
                send_sem=ag_send_sems.at[s], recv_sem=ag_recv_sems.at[s],
                device_id=(p,), device_id_type=pl.DeviceIdType.MESH)
            rdma.start()
            rdma.wait()
            off = off - size * bit

        @pl.when(me == 0)
        def _():
            for b in range(N_SBUF):
                pltpu.make_async_remote_copy(
                    src_ref=sbuf.at[b], dst_ref=kv_comm.at[0],
                    send_sem=sc_send_sems.at[b], recv_sem=kv_recv_sems.at[0],
                    device_id=(0,),
                    device_id_type=pl.DeviceIdType.MESH).wait_send()

        acc[...] = accb[...].astype(jnp.float32)
        cp = pltpu.make_async_copy(acc, out_ref.at[0], copy_sem)
        cp.start()
        cp.wait()

    return pl.pallas_call(
        body,
        out_shape=jax.ShapeDtypeStruct((1, SQ, SQ), jnp.float32),
        in_specs=[
            pl.BlockSpec(memory_space=pltpu.VMEM),
            pl.BlockSpec(memory_space=pltpu.VMEM),
            pl.BlockSpec(memory_space=pl.ANY),
            pl.BlockSpec(memory_space=pl.ANY),
            pl.BlockSpec(memory_space=pltpu.VMEM),
        ],
        out_specs=pl.BlockSpec(memory_space=pl.ANY),
        scratch_shapes=[
            pltpu.VMEM((512, HQ_LOCAL, DH), jnp.float32),
            pltpu.VMEM((N_SBUF, SKV, HQ_LOCAL, DH), jnp.bfloat16),
            pltpu.VMEM((3, SKV, HQ_LOCAL, DH), jnp.bfloat16),
            pltpu.VMEM((2, SKV, HQ_LOCAL, DH), jnp.bfloat16),
            pltpu.VMEM((SQ, SQ), jnp.float32),
            pltpu.VMEM((SQ, SQ), jnp.bfloat16),
            pltpu.VMEM((SQ, SQ), jnp.bfloat16),
            pltpu.VMEM((512, SQ), jnp.bfloat16),
            pltpu.VMEM((256, SQ), jnp.bfloat16),
            pltpu.VMEM((128, SQ), jnp.bfloat16),
            pltpu.VMEM((512, SQ), jnp.bfloat16),
            pltpu.VMEM((256, SQ), jnp.bfloat16),
            pltpu.VMEM((128, SQ), jnp.bfloat16),
            pltpu.SemaphoreType.DMA,
            pltpu.SemaphoreType.DMA((N_SBUF,)),
            pltpu.SemaphoreType.DMA((2,)),
            pltpu.SemaphoreType.DMA((3,)),
            pltpu.SemaphoreType.DMA((3,)),
            pltpu.SemaphoreType.DMA((3,)),
            pltpu.SemaphoreType.DMA((3,)),
            pltpu.SemaphoreType.DMA((3,)),
            pltpu.SemaphoreType.DMA((3,)),
        ],
        compiler_params=pltpu.CompilerParams(
            collective_id=0,
            vmem_limit_bytes=100 * 1024 * 1024,
        ),
    )(x, Wq, K_ext, V_ext, Wo)


# device time: 198989 ns/iter; 2.4510x vs baseline; 1.0264x over previous
import jax
import jax.numpy as jnp
from jax import lax
from jax.experimental import pallas as pl
from jax.experimental.pallas import tpu as pltpu

N_DEV = 8
SQ = 1024
SKV = 1024
HQ_LOCAL = 8
DH = 128
BLK = 64
SCALE = 0.08838834764831843
N_SBUF = 4

SCHEDULE = (
    (0, 2, 1, 0),
    (0, 7, 3, 0),
    (1, 5, 4, 0),
    (1, 2, 1, 1),
    (1, 7, 3, 1),
    (1, 6, 4, 1),
    (0, 5, 1, 2),
    (0, 6, 3, 2),
    (0, 4, 4, -1),
    (0, 1, 1, -1),
    (0, 3, 3, -1),
    (1, 4, 4, -1),
    (1, 1, 1, -1),
    (1, 3, 3, -1),
)
RELAYS = {
    1: ((0, 0, 2), (1, 1, 2), (2, 0, 5)),
    3: ((0, 0, 7), (1, 1, 7), (2, 0, 6)),
    4: ((0, 1, 5), (1, 1, 6)),
}


def kernel(x, Wq, K_ext, V_ext, Wo):
    def body(x_ref, wq_ref, k_ref, v_ref, wo_ref, out_ref,
             kstage, sbuf, relay, kv_comm, acc, accb, qsr,
             ss0, ss1, ss2, rs0, rs1, rs2,
             copy_sem, copy_sems, sc_send_sems, kv_recv_sems,
             relay_recv_sems, relay_fwd_sems,
             rs_send_sems, rs_recv_sems, ag_send_sems, ag_recv_sems):
        me = lax.axis_index("i")

        barrier = pltpu.get_barrier_semaphore()
        for d in range(N_DEV):
            @pl.when(me != d)
            def _():
                pl.semaphore_signal(
                    barrier, inc=1, device_id=(d,),
                    device_id_type=pl.DeviceIdType.MESH)
        pl.semaphore_wait(barrier, N_DEV - 1)

        @pl.when(me == 0)
        def _():
            def stage(tgt, t, dst):
                cps = []
                for half in (0, 1):
                    cp = pltpu.make_async_copy(
                        (k_ref if t == 0 else v_ref)
                        .at[0, pl.ds(512 * half, 512),
                            pl.ds(HQ_LOCAL * tgt, HQ_LOCAL), :],
                        kstage.at[half], copy_sems.at[half])
                    cp.start()
                    cps.append(cp)
                for half in (0, 1):
                    cps[half].wait()
                    dst[pl.ds(512 * half, 512)] = (
                        kstage[half].astype(jnp.bfloat16))

            used = set()
            for idx, (t, tgt, hop, rslot) in enumerate(SCHEDULE):
                b = idx % N_SBUF
                if b in used:
                    pltpu.make_async_remote_copy(
                        src_ref=sbuf.at[b], dst_ref=kv_comm.at[0],
                        send_sem=sc_send_sems.at[b],
                        recv_sem=kv_recv_sems.at[0], device_id=(0,),
                        device_id_type=pl.DeviceIdType.MESH).wait_send()
                stage(tgt, t, sbuf.at[b])
                if hop == tgt:
                    dstr, rsem = kv_comm.at[t], kv_recv_sems.at[t]
                else:
                    dstr, rsem = relay.at[rslot], relay_recv_sems.at[rslot]
                rdma = pltpu.make_async_remote_copy(
                    src_ref=sbuf.at[b], dst_ref=dstr,
                    send_sem=sc_send_sems.at[b], recv_sem=rsem,
                    device_id=(hop,), device_id_type=pl.DeviceIdType.MESH)
                rdma.start()
                used.add(b)
            for t in (0, 1):
                stage(0, t, kv_comm.at[t])

        xb = x_ref[0].astype(jnp.bfloat16)
        wqb = wq_ref[...].astype(jnp.bfloat16)
        q = jnp.dot(xb, wqb, preferred_element_type=jnp.float32)
        qsr[...] = (q * SCALE).astype(jnp.bfloat16)
        wob = wo_ref[...].astype(jnp.bfloat16)

        for dev, duties in RELAYS.items():
            @pl.when(me == dev)
            def _(duties=duties):
                for slot, t, ftgt in duties:
                    rx = pltpu.make_async_remote_copy(
                        src_ref=sbuf.at[0], dst_ref=relay.at[slot],
                        send_sem=sc_send_sems.at[0],
                        recv_sem=relay_recv_sems.at[slot],
                        device_id=(0,), device_id_type=pl.DeviceIdType.MESH)
                    rx.wait_recv()
                    fwd = pltpu.make_async_remote_copy(
                        src_ref=relay.at[slot], dst_ref=kv_comm.at[t],
                        send_sem=relay_fwd_sems.at[slot],
                        recv_sem=kv_recv_sems.at[t],
                        device_id=(ftgt,), device_id_type=pl.DeviceIdType.MESH)
                    fwd.start()
                for slot, t, _ in duties:
                    pltpu.make_async_remote_copy(
                        src_ref=relay.at[slot], dst_ref=kv_comm.at[t],
                        send_sem=relay_fwd_sems.at[slot],
                        recv_sem=kv_recv_sems.at[0],
                        device_id=(0,),
                        device_id_type=pl.DeviceIdType.MESH).wait_send()

        @pl.when(me != 0)
        def _():
            for t in (0, 1):
                pltpu.make_async_remote_copy(
                    src_ref=sbuf.at[0], dst_ref=kv_comm.at[t],
                    send_sem=sc_send_sems.at[0], recv_sem=kv_recv_sems.at[t],
                    device_id=(0,),
                    device_id_type=pl.DeviceIdType.MESH).wait_recv()

        def attn_half(row_off):
            rowblk = (lax.broadcasted_iota(jnp.int32, (512, SKV), 0)
                      + row_off) // BLK
            colblk = lax.broadcasted_iota(jnp.int32, (512, SKV), 1) // BLK
            negh = jnp.where(colblk <= rowblk, 0.0, -1e9).astype(jnp.float32)
            outs = []
            for h in range(HQ_LOCAL):
                qh = qsr[pl.ds(row_off, 512), h * DH:(h + 1) * DH]
                kh = kv_comm[0, :, h, :]
                scores = lax.dot_general(
                    qh, kh, (((1,), (1,)), ((), ())),
                    preferred_element_type=jnp.float32) + negh
                e = jnp.exp(scores)
                s = jnp.sum(e, axis=1, keepdims=True)
                vh = kv_comm[1, :, h, :]
                outs.append(
                    (jnp.dot(e.astype(jnp.bfloat16), vh,
                             preferred_element_type=jnp.float32) / s)
                    .astype(jnp.bfloat16))
            return jnp.concatenate(outs, axis=1)

        z = me // 4
        r = me % 4
        y = r // 2
        xbit = (r % 2) ^ y
        px = me ^ 1
        py = me ^ 3
        pz = me ^ 4

        send_off = 512 * (1 - z)
        keep = 512 * z
        ss0[...] = jnp.dot(attn_half(send_off), wob,
                           preferred_element_type=jnp.float32
                           ).astype(jnp.bfloat16)
        rdma_z = pltpu.make_async_remote_copy(
            src_ref=ss0, dst_ref=rs0,
            send_sem=rs_send_sems.at[0], recv_sem=rs_recv_sems.at[0],
            device_id=(pz,), device_id_type=pl.DeviceIdType.MESH)
        rdma_z.start()
        acc[pl.ds(keep, 512)] = jnp.dot(
            attn_half(keep), wob,
            preferred_element_type=jnp.float32)
        rdma_z.wait()
        acc[pl.ds(keep, 512)] = (
            acc[pl.ds(keep, 512)] + rs0[...].astype(jnp.float32))

        for s, (p, size, bit, sbuf_rs, rbuf) in enumerate(
                ((py, 256, y, ss1, rs1), (px, 128, xbit, ss2, rs2)), start=1):
            send_off = keep + size * (1 - bit)
            keep = keep + size * bit
            sbuf_rs[...] = acc[pl.ds(send_off, size)].astype(jnp.bfloat16)
            rdma = pltpu.make_async_remote_copy(
                src_ref=sbuf_rs, dst_ref=rbuf,
                send_sem=rs_send_sems.at[s], recv_sem=rs_recv_sems.at[s],
                device_id=(p,), device_id_type=pl.DeviceIdType.MESH)
            rdma.start()
            rdma.wait()
            acc[pl.ds(keep, size)] = (
                acc[pl.ds(keep, size)] + rbuf[...].astype(jnp.float32))

        accb[pl.ds(keep, 128)] = acc[pl.ds(keep, 128)].astype(jnp.bfloat16)
        off = keep
        for s, (p, size, bit) in enumerate(
                ((px, 128, xbit), (py, 256, y), (pz, 512, z))):
            rdma = pltpu.make_async_remote_copy(
                src_ref=accb.at[pl.ds(off, size)],
                dst_ref=accb.at[pl.ds(off, size)],
                send_sem=ag_send_sems.at[s], recv_sem=ag_recv_sems.at[s],
                device_id=(p,), device_id_type=pl.DeviceIdType.MESH)
            rdma.start()
            rdma.wait()
            off = off - size * bit

        @pl.when(me == 0)
        def _():
            for b in range(N_SBUF):
                pltpu.make_async_remote_copy(
                    src_ref=sbuf.at[b], dst_ref=kv_comm.at[0],
                    send_sem=sc_send_sems.at[b], recv_sem=kv_recv_sems.at[0],
                    device_id=(0,),
                    device_id_type=pl.DeviceIdType.MESH).wait_send()

        acc[...] = accb[...].astype(jnp.float32)
        cp = pltpu.make_async_copy(acc, out_ref.at[0], copy_sem)
        cp.start()
        cp.wait()

    return pl.pallas_call(
        body,
        out_shape=jax.ShapeDtypeStruct((1, SQ, SQ), jnp.float32),
        in_specs=[
            pl.BlockSpec(memory_space=pltpu.VMEM),
            pl.BlockSpec(memory_space=pltpu.VMEM),
            pl.BlockSpec(memory_space=pl.ANY),
            pl.BlockSpec(memory_space=pl.ANY),
            pl.BlockSpec(memory_space=pltpu.VMEM),
        ],
        out_specs=pl.BlockSpec(memory_space=pl.ANY),
        scratch_shapes=[
            pltpu.VMEM((2, 512, HQ_LOCAL, DH), jnp.float32),
            pltpu.VMEM((N_SBUF, SKV, HQ_LOCAL, DH), jnp.bfloat16),
            pltpu.VMEM((3, SKV, HQ_LOCAL, DH), jnp.bfloat16),
            pltpu.VMEM((2, SKV, HQ_LOCAL, DH), jnp.bfloat16),
            pltpu.VMEM((SQ, SQ), jnp.float32),
            pltpu.VMEM((SQ, SQ), jnp.bfloat16),
            pltpu.VMEM((SQ, SQ), jnp.bfloat16),
            pltpu.VMEM((512, SQ), jnp.bfloat16),
            pltpu.VMEM((256, SQ), jnp.bfloat16),
            pltpu.VMEM((128, SQ), jnp.bfloat16),
            pltpu.VMEM((512, SQ), jnp.bfloat16),
            pltpu.VMEM((256, SQ), jnp.bfloat16),
            pltpu.VMEM((128, SQ), jnp.bfloat16),
            pltpu.SemaphoreType.DMA,
            pltpu.SemaphoreType.DMA((2,)),
            pltpu.SemaphoreType.DMA((N_SBUF,)),
            pltpu.SemaphoreType.DMA((2,)),
            pltpu.SemaphoreType.DMA((3,)),
            pltpu.SemaphoreType.DMA((3,)),
            pltpu.SemaphoreType.DMA((3,)),
            pltpu.SemaphoreType.DMA((3,)),
            pltpu.SemaphoreType.DMA((3,)),
            pltpu.SemaphoreType.DMA((3,)),
        ],
        compiler_params=pltpu.CompilerParams(
            collective_id=0,
            vmem_limit_bytes=100 * 1024 * 1024,
        ),
    )(x, Wq, K_ext, V_ext, Wo)


# device time: 196618 ns/iter; 2.4805x vs baseline; 1.0121x over previous
import jax
import jax.numpy as jnp
from jax import lax
from jax.experimental import pallas as pl
from jax.experimental.pallas import tpu as pltpu

N_DEV = 8
SQ = 1024
SKV = 1024
HQ_LOCAL = 8
DH = 128
BLK = 64
SCALE = 0.08838834764831843
N_SBUF = 4

SCHEDULE = (
    (0, 2, 1, 0),
    (0, 7, 3, 0),
    (1, 5, 4, 0),
    (1, 2, 1, 1),
    (1, 7, 3, 1),
    (1, 6, 4, 1),
    (0, 5, 1, 2),
    (0, 6, 3, 2),
    (0, 4, 4, -1),
    (0, 1, 1, -1),
    (0, 3, 3, -1),
    (1, 4, 4, -1),
    (1, 1, 1, -1),
    (1, 3, 3, -1),
)
RELAYS = {
    1: ((0, 0, 2), (1, 1, 2), (2, 0, 5)),
    3: ((0, 0, 7), (1, 1, 7), (2, 0, 6)),
    4: ((0, 1, 5), (1, 1, 6)),
}


def kernel(x, Wq, K_ext, V_ext, Wo):
    def body(x_ref, wq_ref, k_ref, v_ref, wo_ref, out_ref,
             kstage, sbuf, relay, kv_comm, acc, accb, qsr,
             ss0, ss1, ss2, rs0, rs1, rs2,
             copy_sem, copy_sems, sc_send_sems, kv_recv_sems,
             relay_recv_sems, relay_fwd_sems,
             rs_send_sems, rs_recv_sems, ag_send_sems, ag_recv_sems):
        me = lax.axis_index("i")

        barrier = pltpu.get_barrier_semaphore()
        for d in range(N_DEV):
            @pl.when(me != d)
            def _():
                pl.semaphore_signal(
                    barrier, inc=1, device_id=(d,),
                    device_id_type=pl.DeviceIdType.MESH)
        pl.semaphore_wait(barrier, N_DEV - 1)

        @pl.when(me == 0)
        def _():
            def stage(tgt, t, dst):
                cps = []
                for half in (0, 1):
                    cp = pltpu.make_async_copy(
                        (k_ref if t == 0 else v_ref)
                        .at[0, pl.ds(512 * half, 512),
                            pl.ds(HQ_LOCAL * tgt, HQ_LOCAL), :],
                        kstage.at[half], copy_sems.at[half])
                    cp.start()
                    cps.append(cp)
                for half in (0, 1):
                    cps[half].wait()
                    dst[pl.ds(512 * half, 512)] = (
                        kstage[half].astype(jnp.bfloat16))

            used = set()
            for idx, (t, tgt, hop, rslot) in enumerate(SCHEDULE):
                b = idx % N_SBUF
                if b in used:
                    pltpu.make_async_remote_copy(
                        src_ref=sbuf.at[b], dst_ref=kv_comm.at[0],
                        send_sem=sc_send_sems.at[b],
                        recv_sem=kv_recv_sems.at[0], device_id=(0,),
                        device_id_type=pl.DeviceIdType.MESH).wait_send()
                stage(tgt, t, sbuf.at[b])
                if hop == tgt:
                    dstr, rsem = kv_comm.at[t], kv_recv_sems.at[t]
                else:
                    dstr, rsem = relay.at[rslot], relay_recv_sems.at[rslot]
                rdma = pltpu.make_async_remote_copy(
                    src_ref=sbuf.at[b], dst_ref=dstr,
                    send_sem=sc_send_sems.at[b], recv_sem=rsem,
                    device_id=(hop,), device_id_type=pl.DeviceIdType.MESH)
                rdma.start()
                used.add(b)
            for t in (0, 1):
                stage(0, t, kv_comm.at[t])

        xb = x_ref[0].astype(jnp.bfloat16)
        wqb = wq_ref[...].astype(jnp.bfloat16)
        q = jnp.dot(xb, wqb, preferred_element_type=jnp.float32)
        qsr[...] = (q * SCALE).astype(jnp.bfloat16)
        wob = wo_ref[...].astype(jnp.bfloat16)

        for dev, duties in RELAYS.items():
            @pl.when(me == dev)
            def _(duties=duties):
                for slot, t, ftgt in duties:
                    rx = pltpu.make_async_remote_copy(
                        src_ref=sbuf.at[0], dst_ref=relay.at[slot],
                        send_sem=sc_send_sems.at[0],
                        recv_sem=relay_recv_sems.at[slot],
                        device_id=(0,), device_id_type=pl.DeviceIdType.MESH)
                    rx.wait_recv()
                    fwd = pltpu.make_async_remote_copy(
                        src_ref=relay.at[slot], dst_ref=kv_comm.at[t],
                        send_sem=relay_fwd_sems.at[slot],
                        recv_sem=kv_recv_sems.at[t],
                        device_id=(ftgt,), device_id_type=pl.DeviceIdType.MESH)
                    fwd.start()
                for slot, t, _ in duties:
                    pltpu.make_async_remote_copy(
                        src_ref=relay.at[slot], dst_ref=kv_comm.at[t],
                        send_sem=relay_fwd_sems.at[slot],
                        recv_sem=kv_recv_sems.at[0],
                        device_id=(0,),
                        device_id_type=pl.DeviceIdType.MESH).wait_send()

        @pl.when(me != 0)
        def _():
            for t in (0, 1):
                pltpu.make_async_remote_copy(
                    src_ref=sbuf.at[0], dst_ref=kv_comm.at[t],
                    send_sem=sc_send_sems.at[0], recv_sem=kv_recv_sems.at[t],
                    device_id=(0,),
                    device_id_type=pl.DeviceIdType.MESH).wait_recv()

        def attn_half(row_off):
            rowblk = (lax.broadcasted_iota(jnp.int32, (512, SKV), 0)
                      + row_off) // BLK
            colblk = lax.broadcasted_iota(jnp.int32, (512, SKV), 1) // BLK
            negh = jnp.where(colblk <= rowblk, 0.0, -1e9).astype(jnp.float32)
            outs = []
            for h in range(HQ_LOCAL):
                qh = qsr[pl.ds(row_off, 512), h * DH:(h + 1) * DH]
                kh = kv_comm[0, :, h, :].astype(jnp.bfloat16)
                scores = lax.dot_general(
                    qh, kh, (((1,), (1,)), ((), ())),
                    preferred_element_type=jnp.float32) + negh
                e = jnp.exp(scores)
                s = jnp.sum(e, axis=1, keepdims=True)
                vh = kv_comm[1, :, h, :].astype(jnp.bfloat16)
                outs.append(
                    (jnp.dot(e.astype(jnp.bfloat16), vh,
                             preferred_element_type=jnp.float32) / s)
                    .astype(jnp.bfloat16))
            return jnp.concatenate(outs, axis=1)

        z = me // 4
        r = me % 4
        y = r // 2
        xbit = (r % 2) ^ y
        px = me ^ 1
        py = me ^ 3
        pz = me ^ 4

        send_off = 512 * (1 - z)
        keep = 512 * z
        ss0[...] = jnp.dot(attn_half(send_off), wob,
                           preferred_element_type=jnp.float32
                           ).astype(jnp.bfloat16)
        rdma_z = pltpu.make_async_remote_copy(
            src_ref=ss0, dst_ref=rs0,
            send_sem=rs_send_sems.at[0], recv_sem=rs_recv_sems.at[0],
            device_id=(pz,), device_id_type=pl.DeviceIdType.MESH)
        rdma_z.start()
        acc[pl.ds(keep, 512)] = jnp.dot(
            attn_half(keep), wob,
            preferred_element_type=jnp.float32)
        rdma_z.wait()
        acc[pl.ds(keep, 512)] = (
            acc[pl.ds(keep, 512)] + rs0[...].astype(jnp.float32))

        for s, (p, size, bit, sbuf_rs, rbuf) in enumerate(
                ((py, 256, y, ss1, rs1), (px, 128, xbit, ss2, rs2)), start=1):
            send_off = keep + size * (1 - bit)
            keep = keep + size * bit
            sbuf_rs[...] = acc[pl.ds(send_off, size)].astype(jnp.bfloat16)
            rdma = pltpu.make_async_remote_copy(
                src_ref=sbuf_rs, dst_ref=rbuf,
                send_sem=rs_send_sems.at[s], recv_sem=rs_recv_sems.at[s],
                device_id=(p,), device_id_type=pl.DeviceIdType.MESH)
            rdma.start()
            rdma.wait()
            acc[pl.ds(keep, size)] = (
                acc[pl.ds(keep, size)] + rbuf[...].astype(jnp.float32))

        accb[pl.ds(keep, 128)] = acc[pl.ds(keep, 128)].astype(jnp.bfloat16)
        off = keep
        for s, (p, size, bit) in enumerate(
                ((px, 128, xbit), (py, 256, y), (pz, 512, z))):
            rdma = pltpu.make_async_remote_copy(
                src_ref=accb.at[pl.ds(off, size)],
                dst_ref=accb.at[pl.ds(off, size)],
                send_sem=ag_send_sems.at[s], recv_sem=ag_recv_sems.at[s],
                device_id=(p,), device_id_type=pl.DeviceIdType.MESH)
            rdma.start()
            rdma.wait()
            off = off - size * bit

        @pl.when(me == 0)
        def _():
            for b in range(N_SBUF):
                pltpu.make_async_remote_copy(
                    src_ref=sbuf.at[b], dst_ref=kv_comm.at[0],
                    send_sem=sc_send_sems.at[b], recv_sem=kv_recv_sems.at[0],
                    device_id=(0,),
                    device_id_type=pl.DeviceIdType.MESH).wait_send()

        cp = pltpu.make_async_copy(accb, out_ref.at[0], copy_sem)
        cp.start()
        cp.wait()

    return pl.pallas_call(
        body,
        out_shape=jax.ShapeDtypeStruct((1, SQ, SQ), jnp.bfloat16),
        in_specs=[
            pl.BlockSpec(memory_space=pltpu.VMEM),
            pl.BlockSpec(memory_space=pltpu.VMEM),
            pl.BlockSpec(memory_space=pl.ANY),
            pl.BlockSpec(memory_space=pl.ANY),
            pl.BlockSpec(memory_space=pltpu.VMEM),
        ],
        out_specs=pl.BlockSpec(memory_space=pl.ANY),
        scratch_shapes=[
            pltpu.VMEM((2, 512, HQ_LOCAL, DH), jnp.float32),
            pltpu.VMEM((N_SBUF, SKV, HQ_LOCAL, DH), jnp.bfloat16),
            pltpu.VMEM((3, SKV, HQ_LOCAL, DH), jnp.bfloat16),
            pltpu.VMEM((2, SKV, HQ_LOCAL, DH), jnp.bfloat16),
            pltpu.VMEM((SQ, SQ), jnp.float32),
            pltpu.VMEM((SQ, SQ), jnp.bfloat16),
            pltpu.VMEM((SQ, SQ), jnp.bfloat16),
            pltpu.VMEM((512, SQ), jnp.bfloat16),
            pltpu.VMEM((256, SQ), jnp.bfloat16),
            pltpu.VMEM((128, SQ), jnp.bfloat16),
            pltpu.VMEM((512, SQ), jnp.bfloat16),
            pltpu.VMEM((256, SQ), jnp.bfloat16),
            pltpu.VMEM((128, SQ), jnp.bfloat16),
            pltpu.SemaphoreType.DMA,
            pltpu.SemaphoreType.DMA((2,)),
            pltpu.SemaphoreType.DMA((N_SBUF,)),
            pltpu.SemaphoreType.DMA((2,)),
            pltpu.SemaphoreType.DMA((3,)),
            pltpu.SemaphoreType.DMA((3,)),
            pltpu.SemaphoreType.DMA((3,)),
            pltpu.SemaphoreType.DMA((3,)),
            pltpu.SemaphoreType.DMA((3,)),
            pltpu.SemaphoreType.DMA((3,)),
        ],
        compiler_params=pltpu.CompilerParams(
            collective_id=0,
            vmem_limit_bytes=100 * 1024 * 1024,
        ),
    )(x, Wq, K_ext, V_ext, Wo)
